# Initial kernel scaffold; baseline (speedup 1.0000x reference)
import jax
import jax.numpy as jnp
from jax.experimental import pallas as pl


def kernel(out_mu, out_sigma, out_pi, y, pre_obs):
    raise NotImplementedError("write your pallas kernel here")


# speedup vs baseline: 2.5352x; 2.5352x over previous
import jax
import jax.numpy as jnp
from jax import lax
from jax.experimental import pallas as pl

_EPS = 1e-6


def _body(mu_ref, sg_ref, y_ref, pit_ref, sel_ade_ref, sel_fde_ref,
          reg_ref, cls_ref):
    K, Bb, T2 = mu_ref.shape
    T = T2 // 2
    mu = mu_ref[...]
    y = y_ref[...]
    d = mu - y[None, :, :]
    d2 = d * d
    P = (lax.broadcasted_iota(jnp.int32, (T2, T), 0) // 2 ==
         lax.broadcasted_iota(jnp.int32, (T2, T), 1)).astype(jnp.float32)
    ps = lax.dot_general(d2.reshape(K * Bb, T2), P, (((1,), (0,)), ((), ())),
                         preferred_element_type=jnp.float32,
                         precision=lax.Precision.HIGHEST)
    dist = jnp.sqrt(ps).reshape(K, Bb, T)
    l2 = jnp.sum(dist, axis=-1)
    dfde = dist[:, :, T - 1]

    kio = lax.broadcasted_iota(jnp.int32, (K, Bb), 0)
    minv = jnp.min(l2, axis=0)
    best = jnp.min(jnp.where(l2 == minv[None], kio, K), axis=0)
    mask = (kio == best[None]).astype(jnp.float32)
    minf = jnp.min(dfde, axis=0)
    bestf = jnp.min(jnp.where(dfde == minf[None], kio, K), axis=0)
    maskf = (kio == bestf[None]).astype(jnp.float32)

    sg = sg_ref[...]
    sel_mu = jnp.sum(mask[:, :, None] * mu, axis=0)
    sel_sg = jnp.sum(mask[:, :, None] * sg, axis=0)
    sel_f = jnp.sum(maskf[:, :, None] * mu, axis=0)
    sel_ade_ref[...] = sel_mu
    sel_fde_ref[...] = sel_f

    sc = jnp.maximum(sel_sg, _EPS)
    nll = jnp.log(2.0 * sc) + jnp.abs(y - sel_mu) / sc
    reg_part = jnp.sum(nll)

    z = l2 * (-1.0 / T)
    zm = jnp.max(z, axis=0)
    ez = jnp.exp(z - zm[None])
    st = ez / jnp.sum(ez, axis=0)[None]
    pit = pit_ref[...]
    pm = jnp.max(pit, axis=0)
    lse = jnp.log(jnp.sum(jnp.exp(pit - pm[None]), axis=0)) + pm
    ce = jnp.sum(st * (lse[None] - pit), axis=0)
    cls_part = jnp.sum(ce)

    @pl.when(pl.program_id(0) == 0)
    def _init():
        reg_ref[...] = jnp.zeros_like(reg_ref)
        cls_ref[...] = jnp.zeros_like(cls_ref)

    reg_ref[...] = reg_ref[...] + jnp.reshape(reg_part, (1, 1))
    cls_ref[...] = cls_ref[...] + jnp.reshape(cls_part, (1, 1))


def _run(mu2, sg2, yb, pit, K, B, T2, Bb, interpret=False):
    return pl.pallas_call(
        _body,
        grid=(B // Bb,),
        in_specs=[
            pl.BlockSpec((K, Bb, T2), lambda i: (0, i, 0)),
            pl.BlockSpec((K, Bb, T2), lambda i: (0, i, 0)),
            pl.BlockSpec((Bb, T2), lambda i: (i, 0)),
            pl.BlockSpec((K, Bb), lambda i: (0, i)),
        ],
        out_specs=[
            pl.BlockSpec((Bb, T2), lambda i: (i, 0)),
            pl.BlockSpec((Bb, T2), lambda i: (i, 0)),
            pl.BlockSpec((1, 1), lambda i: (0, 0)),
            pl.BlockSpec((1, 1), lambda i: (0, 0)),
        ],
        out_shape=[
            jax.ShapeDtypeStruct((B, T2), jnp.float32),
            jax.ShapeDtypeStruct((B, T2), jnp.float32),
            jax.ShapeDtypeStruct((1, 1), jnp.float32),
            jax.ShapeDtypeStruct((1, 1), jnp.float32),
        ],
        interpret=interpret,
    )(mu2, sg2, yb, pit)


def kernel(out_mu, out_sigma, out_pi, y, pre_obs):
    K, B, T, _ = out_mu.shape
    T2 = 2 * T
    mu2 = out_mu.reshape(K, B, T2)
    sg2 = out_sigma.reshape(K, B, T2)
    yb = jnp.transpose(y, (1, 0, 2)).reshape(B, T2)
    pit = jnp.transpose(out_pi, (1, 0))
    Bb = 512 if B % 512 == 0 else B
    sel_ade, sel_fde, reg, cls = _run(mu2, sg2, yb, pit, K, B, T2, Bb)
    loss = reg[0, 0] / (B * T2) + cls[0, 0] / B
    sk = jnp.transpose(sel_ade.reshape(B, T, 2), (1, 0, 2))
    skf = jnp.transpose(sel_fde.reshape(B, T, 2), (1, 0, 2))
    tra_ade = jnp.concatenate([pre_obs, sk], axis=0)
    tra_fde = jnp.concatenate([pre_obs, skf], axis=0)
    return (loss, tra_ade, tra_fde)


# speedup vs baseline: 12.4451x; 4.9090x over previous
import jax
import jax.numpy as jnp
from jax import lax
from jax.experimental import pallas as pl

_EPS = 1e-6


def _body(mu_ref, sg_ref, y_ref, pit_ref, sel_ade_ref, sel_fde_ref,
          reg_ref, cls_ref):
    K, T2, Bb = mu_ref.shape
    T = T2 // 2
    mu = mu_ref[...]
    yt = y_ref[...]
    d = mu - yt[None]
    dists = []
    for t in range(T):
        dx = d[:, 2 * t, :]
        dy = d[:, 2 * t + 1, :]
        dists.append(jnp.sqrt(dx * dx + dy * dy))
    l2 = dists[0]
    for t in range(1, T):
        l2 = l2 + dists[t]
    dfde = dists[T - 1]

    kio = lax.broadcasted_iota(jnp.int32, (K, Bb), 0)
    minv = jnp.min(l2, axis=0)
    best = jnp.min(jnp.where(l2 == minv[None], kio, K), axis=0)
    mask = (kio == best[None]).astype(jnp.float32)
    minf = jnp.min(dfde, axis=0)
    bestf = jnp.min(jnp.where(dfde == minf[None], kio, K), axis=0)
    maskf = (kio == bestf[None]).astype(jnp.float32)

    sg = sg_ref[...]
    sel_mu = jnp.sum(mask[:, None, :] * mu, axis=0)
    sel_sg = jnp.sum(mask[:, None, :] * sg, axis=0)
    sel_f = jnp.sum(maskf[:, None, :] * mu, axis=0)
    sel_ade_ref[...] = sel_mu
    sel_fde_ref[...] = sel_f

    sc = jnp.maximum(sel_sg, _EPS)
    nll = jnp.log(2.0 * sc) + jnp.abs(yt - sel_mu) / sc
    reg_part = jnp.sum(nll)

    z = l2 * (-1.0 / T)
    zm = jnp.max(z, axis=0)
    ez = jnp.exp(z - zm[None])
    st = ez / jnp.sum(ez, axis=0)[None]
    pit = pit_ref[...]
    pm = jnp.max(pit, axis=0)
    lse = jnp.log(jnp.sum(jnp.exp(pit - pm[None]), axis=0)) + pm
    ce = jnp.sum(st * (lse[None] - pit), axis=0)
    cls_part = jnp.sum(ce)

    @pl.when(pl.program_id(0) == 0)
    def _init():
        reg_ref[...] = jnp.zeros_like(reg_ref)
        cls_ref[...] = jnp.zeros_like(cls_ref)

    reg_ref[...] = reg_ref[...] + jnp.reshape(reg_part, (1, 1))
    cls_ref[...] = cls_ref[...] + jnp.reshape(cls_part, (1, 1))


def _run(mu_t, sg_t, y_t, pit, K, B, T2, Bb, interpret=False):
    return pl.pallas_call(
        _body,
        grid=(B // Bb,),
        in_specs=[
            pl.BlockSpec((K, T2, Bb), lambda i: (0, 0, i)),
            pl.BlockSpec((K, T2, Bb), lambda i: (0, 0, i)),
            pl.BlockSpec((T2, Bb), lambda i: (0, i)),
            pl.BlockSpec((K, Bb), lambda i: (0, i)),
        ],
        out_specs=[
            pl.BlockSpec((T2, Bb), lambda i: (0, i)),
            pl.BlockSpec((T2, Bb), lambda i: (0, i)),
            pl.BlockSpec((1, 1), lambda i: (0, 0)),
            pl.BlockSpec((1, 1), lambda i: (0, 0)),
        ],
        out_shape=[
            jax.ShapeDtypeStruct((T2, B), jnp.float32),
            jax.ShapeDtypeStruct((T2, B), jnp.float32),
            jax.ShapeDtypeStruct((1, 1), jnp.float32),
            jax.ShapeDtypeStruct((1, 1), jnp.float32),
        ],
        interpret=interpret,
    )(mu_t, sg_t, y_t, pit)


def kernel(out_mu, out_sigma, out_pi, y, pre_obs):
    K, B, T, _ = out_mu.shape
    T2 = 2 * T
    mu_t = jnp.transpose(out_mu.reshape(K, B, T2), (0, 2, 1))
    sg_t = jnp.transpose(out_sigma.reshape(K, B, T2), (0, 2, 1))
    y_t = jnp.transpose(y, (0, 2, 1)).reshape(T2, B)
    pit = jnp.transpose(out_pi, (1, 0))
    Bb = 2048 if B % 2048 == 0 else B
    sel_ade, sel_fde, reg, cls = _run(mu_t, sg_t, y_t, pit, K, B, T2, Bb)
    loss = reg[0, 0] / (B * T2) + cls[0, 0] / B
    sk = jnp.transpose(sel_ade.reshape(T, 2, B), (0, 2, 1))
    skf = jnp.transpose(sel_fde.reshape(T, 2, B), (0, 2, 1))
    tra_ade = jnp.concatenate([pre_obs, sk], axis=0)
    tra_fde = jnp.concatenate([pre_obs, skf], axis=0)
    return (loss, tra_ade, tra_fde)
